# baseline (speedup 1.0000x reference)
import functools

import jax
import jax.numpy as jnp
from jax import lax
from jax.experimental import pallas as pl
from jax.experimental.pallas import tpu as pltpu
from jax.experimental.pallas import tpu_sc as plsc

_NC = 2
_NS = 16
_NW = _NC * _NS
_CHUNK = 64


def _sc_gather_pair(exe_idx, skill_idx, exe_table, skill_table):
    n = exe_idx.shape[0]
    d = exe_table.shape[1]
    per_w = n // _NW
    n_chunks = per_w // _CHUNK
    mesh = plsc.VectorSubcoreMesh(
        core_axis_name="c", subcore_axis_name="s",
        num_cores=_NC, num_subcores=_NS,
    )

    @functools.partial(
        pl.kernel,
        mesh=mesh,
        out_type=[
            jax.ShapeDtypeStruct((n, d), jnp.float32),
            jax.ShapeDtypeStruct((n, d), jnp.float32),
        ],
        scratch_types=[
            pltpu.VMEM((_CHUNK,), jnp.int32),
            pltpu.VMEM((_CHUNK,), jnp.int32),
            pltpu.VMEM((_CHUNK, d), jnp.float32),
            pltpu.VMEM((_CHUNK, d), jnp.float32),
            pltpu.SemaphoreType.DMA,
            pltpu.SemaphoreType.DMA,
        ],
    )
    def gather_kernel(exe_idx_hbm, skill_idx_hbm, exe_tab_hbm, skill_tab_hbm,
                      out_exe_hbm, out_skill_hbm,
                      idx_e, idx_s, rows_e, rows_s, sem_e, sem_s):
        wid = lax.axis_index("s") * _NC + lax.axis_index("c")
        base = wid * per_w

        def body(c, carry):
            off = base + c * _CHUNK
            pltpu.sync_copy(exe_idx_hbm.at[pl.ds(off, _CHUNK)], idx_e)
            pltpu.sync_copy(skill_idx_hbm.at[pl.ds(off, _CHUNK)], idx_s)
            cp_e = pltpu.async_copy(exe_tab_hbm.at[idx_e], rows_e, sem_e)
            cp_s = pltpu.async_copy(skill_tab_hbm.at[idx_s], rows_s, sem_s)
            cp_e.wait()
            cp_s.wait()
            pltpu.sync_copy(rows_e, out_exe_hbm.at[pl.ds(off, _CHUNK)])
            pltpu.sync_copy(rows_s, out_skill_hbm.at[pl.ds(off, _CHUNK)])
            return carry

        lax.fori_loop(0, n_chunks, body, 0)

    return gather_kernel(exe_idx, skill_idx, exe_table, skill_table)


def _tc_ed_body(bb, in_nlp, exe_id, skill_id, r_id, et,
                pos, W, b, etW, etb, resp, enc_o, dec_o):
    Wv = W[...]
    bv = b[...]
    posv = pos[...]
    respv = resp[...]
    etWv = etW[...]
    etbv = etb[...]
    ones_row = jnp.ones((1, posv.shape[1]), jnp.float32)
    ids2 = (exe_id[...] + skill_id[...]).astype(jnp.float32)
    r2 = r_id[...]
    dn = (((0,), (0,)), ((), ()))
    for j in range(bb):
        ids_bc = lax.dot_general(ids2[j:j + 1, :], ones_row, dn,
                                 preferred_element_type=jnp.float32)
        enc_o[j] = (
            jnp.dot(in_nlp[j], Wv, preferred_element_type=jnp.float32)
            + bv + ids_bc + posv
        )
        rj = r2[j:j + 1, :]
        oh = jnp.concatenate(
            [(rj == t).astype(jnp.float32) for t in range(respv.shape[0])],
            axis=0,
        )
        resp_sel = lax.dot_general(oh, respv, dn,
                                   preferred_element_type=jnp.float32)
        et_bc = jnp.dot(et[j], etWv, preferred_element_type=jnp.float32)
        dec_o[j] = resp_sel + et_bc + etbv + posv


def _tc_out_body(bb, s, out_nlp, gexe, gskill, W, b, out_o):
    Wv = W[...]
    bv = b[...]
    for j in range(bb):
        out_o[j] = (
            jnp.dot(out_nlp[j], Wv, preferred_element_type=jnp.float32)
            + bv + gexe[pl.ds(j * s, s), :] + gskill[pl.ds(j * s, s), :]
        )


def _tc_enc_dec(in_nlp, exe_ids, skill_ids, r_ids, et,
                pos, W, b, etW, etb, resp, bb, interpret=False):
    bsz, s, nlp = in_nlp.shape
    d = W.shape[1]
    grid = (bsz // bb,)
    batch3 = lambda w: pl.BlockSpec((bb, s, w), lambda i: (i, 0, 0))
    batch2 = pl.BlockSpec((bb, s), lambda i: (i, 0))
    full2 = lambda h: pl.BlockSpec((h, d), lambda i: (0, 0))
    return pl.pallas_call(
        functools.partial(_tc_ed_body, bb),
        grid=grid,
        in_specs=[
            batch3(nlp),
            batch2, batch2, batch2, batch3(1),
            full2(s),
            pl.BlockSpec((nlp, d), lambda i: (0, 0)),
            full2(1), full2(1), full2(1), full2(resp.shape[0]),
        ],
        out_specs=[batch3(d), batch3(d)],
        out_shape=[jax.ShapeDtypeStruct((bsz, s, d), jnp.float32)] * 2,
        compiler_params=pltpu.CompilerParams(
            dimension_semantics=("arbitrary",),
        ),
        interpret=interpret,
    )(in_nlp, exe_ids, skill_ids, r_ids, et, pos, W, b, etW, etb, resp)


def _tc_out(out_nlp, g_exe, g_skill, W, b, bb, interpret=False):
    bsz, s, nlp = out_nlp.shape
    d = W.shape[1]
    grid = (bsz // bb,)
    batch3 = lambda w: pl.BlockSpec((bb, s, w), lambda i: (i, 0, 0))
    rows2 = pl.BlockSpec((bb * s, d), lambda i: (i, 0))
    return pl.pallas_call(
        functools.partial(_tc_out_body, bb, s),
        grid=grid,
        in_specs=[
            batch3(nlp), rows2, rows2,
            pl.BlockSpec((nlp, d), lambda i: (0, 0)),
            pl.BlockSpec((1, d), lambda i: (0, 0)),
        ],
        out_specs=[batch3(d)],
        out_shape=[jax.ShapeDtypeStruct((bsz, s, d), jnp.float32)],
        compiler_params=pltpu.CompilerParams(
            dimension_semantics=("arbitrary",),
        ),
        interpret=interpret,
    )(out_nlp, g_exe, g_skill, W, b)[0]


def kernel(input_nlp_embedding, input_exercise, input_skill, input_r,
           in_elapsed_time, output_nlp_embedding, out_exercise, out_skill,
           exercise_table, skill_table, response_table, pos_table,
           nlp_W, nlp_b, et_W, et_b):
    b_dim, s_dim, nlp = input_nlp_embedding.shape
    d = nlp_W.shape[1]
    n = b_dim * s_dim

    g_exe, g_skill = _sc_gather_pair(
        out_exercise.reshape(n), out_skill.reshape(n),
        exercise_table, skill_table,
    )

    enc, dec = _tc_enc_dec(
        input_nlp_embedding, input_exercise, input_skill, input_r,
        in_elapsed_time, pos_table,
        nlp_W, nlp_b.reshape(1, d), et_W, et_b.reshape(1, d),
        response_table, bb=16,
    )
    outp = _tc_out(output_nlp_embedding, g_exe, g_skill,
                   nlp_W, nlp_b.reshape(1, d), bb=16)
    return (enc, dec, outp)


# speedup vs baseline: 1.9662x; 1.9662x over previous
import functools

import jax
import jax.numpy as jnp
from jax import lax
from jax.experimental import pallas as pl
from jax.experimental.pallas import tpu as pltpu
from jax.experimental.pallas import tpu_sc as plsc

_NC = 2
_NS = 16
_NW = _NC * _NS
_CHUNK = 64


def _sc_gather_pair(exe_idx, skill_idx, exe_table, skill_table):
    n = exe_idx.shape[0]
    d = exe_table.shape[1]
    per_w = n // _NW
    n_chunks = per_w // _CHUNK
    mesh = plsc.VectorSubcoreMesh(
        core_axis_name="c", subcore_axis_name="s",
        num_cores=_NC, num_subcores=_NS,
    )

    @functools.partial(
        pl.kernel,
        mesh=mesh,
        out_type=[
            jax.ShapeDtypeStruct((n, d), jnp.float32),
            jax.ShapeDtypeStruct((n, d), jnp.float32),
        ],
        scratch_types=[
            pltpu.VMEM((_CHUNK,), jnp.int32),
            pltpu.VMEM((_CHUNK,), jnp.int32),
            pltpu.VMEM((_CHUNK, d), jnp.float32),
            pltpu.VMEM((_CHUNK, d), jnp.float32),
            pltpu.SemaphoreType.DMA,
            pltpu.SemaphoreType.DMA,
        ],
    )
    def gather_kernel(exe_idx_hbm, skill_idx_hbm, exe_tab_hbm, skill_tab_hbm,
                      out_exe_hbm, out_skill_hbm,
                      idx_e, idx_s, rows_e, rows_s, sem_e, sem_s):
        wid = lax.axis_index("s") * _NC + lax.axis_index("c")
        base = wid * per_w

        def body(c, carry):
            off = base + c * _CHUNK
            pltpu.sync_copy(exe_idx_hbm.at[pl.ds(off, _CHUNK)], idx_e)
            pltpu.sync_copy(skill_idx_hbm.at[pl.ds(off, _CHUNK)], idx_s)
            cp_e = pltpu.async_copy(exe_tab_hbm.at[idx_e], rows_e, sem_e)
            cp_s = pltpu.async_copy(skill_tab_hbm.at[idx_s], rows_s, sem_s)
            cp_e.wait()
            cp_s.wait()
            pltpu.sync_copy(rows_e, out_exe_hbm.at[pl.ds(off, _CHUNK)])
            pltpu.sync_copy(rows_s, out_skill_hbm.at[pl.ds(off, _CHUNK)])
            return carry

        lax.fori_loop(0, n_chunks, body, 0)

    return gather_kernel(exe_idx, skill_idx, exe_table, skill_table)


def _tc_ed_body(x, exe_id, skill_id, r_id, et, pos, W, b, etW, etb, resp,
                enc_o, dec_o):
    Wv = W[...]
    posr = pos[0]
    ids_f = (exe_id[...] + skill_id[...]).astype(jnp.float32)
    enc_o[...] = (
        jnp.dot(x[...], Wv, preferred_element_type=jnp.float32)
        + b[...] + ids_f + posr
    )
    r = r_id[...]
    respv = resp[...]
    resp_sel = jnp.where(
        r == 0, respv[0:1, :],
        jnp.where(r == 1, respv[1:2, :], respv[2:3, :]))
    dec_o[...] = resp_sel + et[...] * etW[...] + etb[...] + posr


def _tc_out_body(y, gexe, gskill, W, b, out_o):
    out_o[...] = (
        jnp.dot(y[...], W[...], preferred_element_type=jnp.float32)
        + b[...] + gexe[...] + gskill[...]
    )


def _tc_enc_dec(x, exe_ids, skill_ids, r_ids, et, pos, W, b, etW, etb, resp,
                rows, interpret=False):
    n, nlp = x.shape
    d = W.shape[1]
    grid = (n // rows,)
    row_spec = lambda w: pl.BlockSpec((rows, w), lambda i: (i, 0))
    full = lambda h, w: pl.BlockSpec((h, w), lambda i: (0, 0))
    return pl.pallas_call(
        _tc_ed_body,
        grid=grid,
        in_specs=[
            row_spec(nlp),
            row_spec(1), row_spec(1), row_spec(1), row_spec(1),
            pl.BlockSpec((1, 1, d), lambda i: (i, 0, 0)),
            full(nlp, d), full(1, d), full(1, d), full(1, d),
            full(resp.shape[0], d),
        ],
        out_specs=[row_spec(d), row_spec(d)],
        out_shape=[jax.ShapeDtypeStruct((n, d), jnp.float32)] * 2,
        compiler_params=pltpu.CompilerParams(
            dimension_semantics=("arbitrary",),
        ),
        interpret=interpret,
    )(x, exe_ids, skill_ids, r_ids, et, pos, W, b, etW, etb, resp)


def _tc_out(y, g_exe, g_skill, W, b, rows, interpret=False):
    n, nlp = y.shape
    d = W.shape[1]
    grid = (n // rows,)
    row_spec = lambda w: pl.BlockSpec((rows, w), lambda i: (i, 0))
    return pl.pallas_call(
        _tc_out_body,
        grid=grid,
        in_specs=[
            row_spec(nlp), row_spec(d), row_spec(d),
            pl.BlockSpec((nlp, d), lambda i: (0, 0)),
            pl.BlockSpec((1, d), lambda i: (0, 0)),
        ],
        out_specs=[row_spec(d)],
        out_shape=[jax.ShapeDtypeStruct((n, d), jnp.float32)],
        compiler_params=pltpu.CompilerParams(
            dimension_semantics=("arbitrary",),
        ),
        interpret=interpret,
    )(y, g_exe, g_skill, W, b)[0]


def kernel(input_nlp_embedding, input_exercise, input_skill, input_r,
           in_elapsed_time, output_nlp_embedding, out_exercise, out_skill,
           exercise_table, skill_table, response_table, pos_table,
           nlp_W, nlp_b, et_W, et_b):
    b_dim, s_dim, nlp = input_nlp_embedding.shape
    d = nlp_W.shape[1]
    n = b_dim * s_dim

    x_in = jnp.transpose(input_nlp_embedding, (1, 0, 2)).reshape(n, nlp)
    x_out = jnp.transpose(output_nlp_embedding, (1, 0, 2)).reshape(n, nlp)
    exe_t = jnp.transpose(input_exercise).reshape(n, 1)
    skill_t = jnp.transpose(input_skill).reshape(n, 1)
    r_t = jnp.transpose(input_r).reshape(n, 1)
    et_t = jnp.transpose(in_elapsed_time, (1, 0, 2)).reshape(n, 1)
    oexe_t = jnp.transpose(out_exercise).reshape(n)
    oskill_t = jnp.transpose(out_skill).reshape(n)

    g_exe, g_skill = _sc_gather_pair(
        oexe_t, oskill_t, exercise_table, skill_table,
    )

    enc2, dec2 = _tc_enc_dec(
        x_in, exe_t, skill_t, r_t, et_t, pos_table.reshape(s_dim, 1, d),
        nlp_W, nlp_b.reshape(1, d), et_W, et_b.reshape(1, d),
        response_table, rows=b_dim,
    )
    out2 = _tc_out(x_out, g_exe, g_skill, nlp_W, nlp_b.reshape(1, d),
                   rows=b_dim)

    def back(a2):
        return jnp.transpose(a2.reshape(s_dim, b_dim, d), (1, 0, 2))

    return (back(enc2), back(dec2), back(out2))


# speedup vs baseline: 2.5397x; 1.2917x over previous
import functools

import jax
import jax.numpy as jnp
from jax import lax
from jax.experimental import pallas as pl
from jax.experimental.pallas import tpu as pltpu
from jax.experimental.pallas import tpu_sc as plsc

_NC = 2
_NS = 16
_NW = _NC * _NS
_CHUNK = 64


def _sc_gather_pair(exe_idx, skill_idx, exe_table, skill_table):
    n = exe_idx.shape[0]
    d = exe_table.shape[1]
    per_w = n // _NW
    n_chunks = per_w // _CHUNK
    mesh = plsc.VectorSubcoreMesh(
        core_axis_name="c", subcore_axis_name="s",
        num_cores=_NC, num_subcores=_NS,
    )

    @functools.partial(
        pl.kernel,
        mesh=mesh,
        out_type=[
            jax.ShapeDtypeStruct((n, d), jnp.float32),
            jax.ShapeDtypeStruct((n, d), jnp.float32),
        ],
        scratch_types=[
            pltpu.VMEM((_CHUNK,), jnp.int32),
            pltpu.VMEM((_CHUNK,), jnp.int32),
            pltpu.VMEM((_CHUNK, d), jnp.float32),
            pltpu.VMEM((_CHUNK, d), jnp.float32),
            pltpu.SemaphoreType.DMA,
            pltpu.SemaphoreType.DMA,
        ],
    )
    def gather_kernel(exe_idx_hbm, skill_idx_hbm, exe_tab_hbm, skill_tab_hbm,
                      out_exe_hbm, out_skill_hbm,
                      idx_e, idx_s, rows_e, rows_s, sem_e, sem_s):
        wid = lax.axis_index("s") * _NC + lax.axis_index("c")
        base = wid * per_w

        def body(c, carry):
            off = base + c * _CHUNK
            pltpu.sync_copy(exe_idx_hbm.at[pl.ds(off, _CHUNK)], idx_e)
            pltpu.sync_copy(skill_idx_hbm.at[pl.ds(off, _CHUNK)], idx_s)
            cp_e = pltpu.async_copy(exe_tab_hbm.at[idx_e], rows_e, sem_e)
            cp_s = pltpu.async_copy(skill_tab_hbm.at[idx_s], rows_s, sem_s)
            cp_e.wait()
            cp_s.wait()
            pltpu.sync_copy(rows_e, out_exe_hbm.at[pl.ds(off, _CHUNK)])
            pltpu.sync_copy(rows_s, out_skill_hbm.at[pl.ds(off, _CHUNK)])
            return carry

        lax.fori_loop(0, n_chunks, body, 0)

    return gather_kernel(exe_idx, skill_idx, exe_table, skill_table)


def _tc_ed_body(x, exe_id, skill_id, r_id, et, pos, W, b, etW, etb, resp,
                enc_o, dec_o):
    Wv = W[...]
    posr = pos[0]
    d = posr.shape[1]
    dn = (((0,), (0,)), ((), ()))
    ones_row = jnp.ones((1, d), jnp.float32)
    ids_row = (exe_id[0] + skill_id[0]).astype(jnp.float32)
    ids_bc = lax.dot_general(ids_row, ones_row, dn,
                             preferred_element_type=jnp.float32)
    enc_o[...] = (
        jnp.dot(x[...], Wv, preferred_element_type=jnp.float32)
        + b[...] + ids_bc + posr
    )
    r_row = r_id[0]
    respv = resp[...]
    oh = jnp.concatenate(
        [(r_row == t).astype(jnp.float32) for t in range(respv.shape[0])],
        axis=0,
    )
    resp_sel = lax.dot_general(oh, respv, dn,
                               preferred_element_type=jnp.float32)
    et_bc = lax.dot_general(et[0], etW[...], dn,
                            preferred_element_type=jnp.float32)
    dec_o[...] = resp_sel + et_bc + etb[...] + posr


def _tc_out_body(y, gexe, gskill, W, b, out_o):
    out_o[...] = (
        jnp.dot(y[...], W[...], preferred_element_type=jnp.float32)
        + b[...] + gexe[...] + gskill[...]
    )


def _tc_enc_dec(x, exe_ids, skill_ids, r_ids, et, pos, W, b, etW, etb, resp,
                rows, interpret=False):
    n, nlp = x.shape
    d = W.shape[1]
    grid = (n // rows,)
    row_spec = lambda w: pl.BlockSpec((rows, w), lambda i: (i, 0))
    full = lambda h, w: pl.BlockSpec((h, w), lambda i: (0, 0))
    return pl.pallas_call(
        _tc_ed_body,
        grid=grid,
        in_specs=[
            row_spec(nlp),
            pl.BlockSpec((1, 1, rows), lambda i: (i, 0, 0)),
            pl.BlockSpec((1, 1, rows), lambda i: (i, 0, 0)),
            pl.BlockSpec((1, 1, rows), lambda i: (i, 0, 0)),
            pl.BlockSpec((1, 1, rows), lambda i: (i, 0, 0)),
            pl.BlockSpec((1, 1, d), lambda i: (i, 0, 0)),
            full(nlp, d), full(1, d), full(1, d), full(1, d),
            full(resp.shape[0], d),
        ],
        out_specs=[row_spec(d), row_spec(d)],
        out_shape=[jax.ShapeDtypeStruct((n, d), jnp.float32)] * 2,
        compiler_params=pltpu.CompilerParams(
            dimension_semantics=("arbitrary",),
        ),
        interpret=interpret,
    )(x, exe_ids, skill_ids, r_ids, et, pos, W, b, etW, etb, resp)


def _tc_out(y, g_exe, g_skill, W, b, rows, interpret=False):
    n, nlp = y.shape
    d = W.shape[1]
    grid = (n // rows,)
    row_spec = lambda w: pl.BlockSpec((rows, w), lambda i: (i, 0))
    return pl.pallas_call(
        _tc_out_body,
        grid=grid,
        in_specs=[
            row_spec(nlp), row_spec(d), row_spec(d),
            pl.BlockSpec((nlp, d), lambda i: (0, 0)),
            pl.BlockSpec((1, d), lambda i: (0, 0)),
        ],
        out_specs=[row_spec(d)],
        out_shape=[jax.ShapeDtypeStruct((n, d), jnp.float32)],
        compiler_params=pltpu.CompilerParams(
            dimension_semantics=("arbitrary",),
        ),
        interpret=interpret,
    )(y, g_exe, g_skill, W, b)[0]


def kernel(input_nlp_embedding, input_exercise, input_skill, input_r,
           in_elapsed_time, output_nlp_embedding, out_exercise, out_skill,
           exercise_table, skill_table, response_table, pos_table,
           nlp_W, nlp_b, et_W, et_b):
    b_dim, s_dim, nlp = input_nlp_embedding.shape
    d = nlp_W.shape[1]
    n = b_dim * s_dim

    x_in = jnp.transpose(input_nlp_embedding, (1, 0, 2)).reshape(n, nlp)
    x_out = jnp.transpose(output_nlp_embedding, (1, 0, 2)).reshape(n, nlp)
    exe_t = jnp.transpose(input_exercise).reshape(s_dim, 1, b_dim)
    skill_t = jnp.transpose(input_skill).reshape(s_dim, 1, b_dim)
    r_t = jnp.transpose(input_r).reshape(s_dim, 1, b_dim)
    et_t = jnp.transpose(in_elapsed_time, (1, 2, 0))
    oexe_t = jnp.transpose(out_exercise).reshape(n)
    oskill_t = jnp.transpose(out_skill).reshape(n)

    g_exe, g_skill = _sc_gather_pair(
        oexe_t, oskill_t, exercise_table, skill_table,
    )

    enc2, dec2 = _tc_enc_dec(
        x_in, exe_t, skill_t, r_t, et_t, pos_table.reshape(s_dim, 1, d),
        nlp_W, nlp_b.reshape(1, d), et_W, et_b.reshape(1, d),
        response_table, rows=b_dim,
    )
    out2 = _tc_out(x_out, g_exe, g_skill, nlp_W, nlp_b.reshape(1, d),
                   rows=b_dim)

    def back(a2):
        return jnp.transpose(a2.reshape(s_dim, b_dim, d), (1, 0, 2))

    return (back(enc2), back(dec2), back(out2))


# speedup vs baseline: 2.6743x; 1.0530x over previous
import functools

import jax
import jax.numpy as jnp
from jax import lax
from jax.experimental import pallas as pl
from jax.experimental.pallas import tpu as pltpu
from jax.experimental.pallas import tpu_sc as plsc

_NC = 2
_NS = 16
_NW = _NC * _NS
_CHUNK = 64


def _sc_gather_pair(exe_idx, skill_idx, exe_table, skill_table):
    n = exe_idx.shape[0]
    d = exe_table.shape[1]
    per_w = n // _NW
    n_chunks = per_w // _CHUNK
    mesh = plsc.VectorSubcoreMesh(
        core_axis_name="c", subcore_axis_name="s",
        num_cores=_NC, num_subcores=_NS,
    )

    @functools.partial(
        pl.kernel,
        mesh=mesh,
        out_type=[
            jax.ShapeDtypeStruct((n, d), jnp.float32),
            jax.ShapeDtypeStruct((n, d), jnp.float32),
        ],
        scratch_types=[
            pltpu.VMEM((_CHUNK,), jnp.int32),
            pltpu.VMEM((_CHUNK,), jnp.int32),
            pltpu.VMEM((_CHUNK, d), jnp.float32),
            pltpu.VMEM((_CHUNK, d), jnp.float32),
            pltpu.SemaphoreType.DMA,
            pltpu.SemaphoreType.DMA,
        ],
    )
    def gather_kernel(exe_idx_hbm, skill_idx_hbm, exe_tab_hbm, skill_tab_hbm,
                      out_exe_hbm, out_skill_hbm,
                      idx_e, idx_s, rows_e, rows_s, sem_e, sem_s):
        wid = lax.axis_index("s") * _NC + lax.axis_index("c")
        base = wid * per_w

        def body(c, carry):
            off = base + c * _CHUNK
            pltpu.sync_copy(exe_idx_hbm.at[pl.ds(off, _CHUNK)], idx_e)
            pltpu.sync_copy(skill_idx_hbm.at[pl.ds(off, _CHUNK)], idx_s)
            cp_e = pltpu.async_copy(exe_tab_hbm.at[idx_e], rows_e, sem_e)
            cp_s = pltpu.async_copy(skill_tab_hbm.at[idx_s], rows_s, sem_s)
            cp_e.wait()
            cp_s.wait()
            pltpu.sync_copy(rows_e, out_exe_hbm.at[pl.ds(off, _CHUNK)])
            pltpu.sync_copy(rows_s, out_skill_hbm.at[pl.ds(off, _CHUNK)])
            return carry

        lax.fori_loop(0, n_chunks, body, 0)

    return gather_kernel(exe_idx, skill_idx, exe_table, skill_table)


def _tc_ed_body(x, exe_id, skill_id, r_id, et, pos, W, b, etW, etb, resp,
                enc_o, dec_o):
    Wv = W[...]
    posr = pos[0]
    d = posr.shape[1]
    dn = (((0,), (0,)), ((), ()))
    ones_row = jnp.ones((1, d), jnp.float32)
    ids_row = (exe_id[0] + skill_id[0]).astype(jnp.float32)
    ids_bc = lax.dot_general(ids_row, ones_row, dn,
                             preferred_element_type=jnp.float32)
    enc_o[...] = (
        jnp.dot(x[...], Wv, preferred_element_type=jnp.float32)
        + b[...] + ids_bc + posr
    )
    r_row = r_id[0]
    respv = resp[...]
    oh = jnp.concatenate(
        [(r_row == t).astype(jnp.float32) for t in range(respv.shape[0])],
        axis=0,
    )
    resp_sel = lax.dot_general(oh, respv, dn,
                               preferred_element_type=jnp.float32)
    et_bc = lax.dot_general(et[0], etW[...], dn,
                            preferred_element_type=jnp.float32)
    dec_o[...] = resp_sel + et_bc + etb[...] + posr


def _tc_out_body(y, gexe, gskill, W, b, out_o):
    out_o[...] = (
        jnp.dot(y[...], W[...], preferred_element_type=jnp.float32)
        + b[...] + gexe[...] + gskill[...]
    )


def _tc_enc_dec(x, exe_ids, skill_ids, r_ids, et, pos, W, b, etW, etb, resp,
                rows, interpret=False):
    n, nlp = x.shape
    d = W.shape[1]
    grid = (n // rows,)
    row_spec = lambda w: pl.BlockSpec((rows, w), lambda i: (i, 0))
    full = lambda h, w: pl.BlockSpec((h, w), lambda i: (0, 0))
    return pl.pallas_call(
        _tc_ed_body,
        grid=grid,
        in_specs=[
            row_spec(nlp),
            pl.BlockSpec((1, 1, rows), lambda i: (i, 0, 0)),
            pl.BlockSpec((1, 1, rows), lambda i: (i, 0, 0)),
            pl.BlockSpec((1, 1, rows), lambda i: (i, 0, 0)),
            pl.BlockSpec((1, 1, rows), lambda i: (i, 0, 0)),
            pl.BlockSpec((1, 1, d), lambda i: (i, 0, 0)),
            full(nlp, d), full(1, d), full(1, d), full(1, d),
            full(resp.shape[0], d),
        ],
        out_specs=[row_spec(d), row_spec(d)],
        out_shape=[jax.ShapeDtypeStruct((n, d), jnp.float32)] * 2,
        compiler_params=pltpu.CompilerParams(
            dimension_semantics=("parallel",),
        ),
        interpret=interpret,
    )(x, exe_ids, skill_ids, r_ids, et, pos, W, b, etW, etb, resp)


def _tc_out(y, g_exe, g_skill, W, b, rows, interpret=False):
    n, nlp = y.shape
    d = W.shape[1]
    grid = (n // rows,)
    row_spec = lambda w: pl.BlockSpec((rows, w), lambda i: (i, 0))
    return pl.pallas_call(
        _tc_out_body,
        grid=grid,
        in_specs=[
            row_spec(nlp), row_spec(d), row_spec(d),
            pl.BlockSpec((nlp, d), lambda i: (0, 0)),
            pl.BlockSpec((1, d), lambda i: (0, 0)),
        ],
        out_specs=[row_spec(d)],
        out_shape=[jax.ShapeDtypeStruct((n, d), jnp.float32)],
        compiler_params=pltpu.CompilerParams(
            dimension_semantics=("parallel",),
        ),
        interpret=interpret,
    )(y, g_exe, g_skill, W, b)[0]


def kernel(input_nlp_embedding, input_exercise, input_skill, input_r,
           in_elapsed_time, output_nlp_embedding, out_exercise, out_skill,
           exercise_table, skill_table, response_table, pos_table,
           nlp_W, nlp_b, et_W, et_b):
    b_dim, s_dim, nlp = input_nlp_embedding.shape
    d = nlp_W.shape[1]
    n = b_dim * s_dim

    x_in = jnp.transpose(input_nlp_embedding, (1, 0, 2)).reshape(n, nlp)
    x_out = jnp.transpose(output_nlp_embedding, (1, 0, 2)).reshape(n, nlp)
    exe_t = jnp.transpose(input_exercise).reshape(s_dim, 1, b_dim)
    skill_t = jnp.transpose(input_skill).reshape(s_dim, 1, b_dim)
    r_t = jnp.transpose(input_r).reshape(s_dim, 1, b_dim)
    et_t = jnp.transpose(in_elapsed_time, (1, 2, 0))
    oexe_t = jnp.transpose(out_exercise).reshape(n)
    oskill_t = jnp.transpose(out_skill).reshape(n)

    g_exe, g_skill = _sc_gather_pair(
        oexe_t, oskill_t, exercise_table, skill_table,
    )

    enc2, dec2 = _tc_enc_dec(
        x_in, exe_t, skill_t, r_t, et_t, pos_table.reshape(s_dim, 1, d),
        nlp_W, nlp_b.reshape(1, d), et_W, et_b.reshape(1, d),
        response_table, rows=b_dim,
    )
    out2 = _tc_out(x_out, g_exe, g_skill, nlp_W, nlp_b.reshape(1, d),
                   rows=2 * b_dim)

    def back(a2):
        return jnp.transpose(a2.reshape(s_dim, b_dim, d), (1, 0, 2))

    return (back(enc2), back(dec2), back(out2))


# speedup vs baseline: 2.7209x; 1.0174x over previous
import functools

import jax
import jax.numpy as jnp
from jax import lax
from jax.experimental import pallas as pl
from jax.experimental.pallas import tpu as pltpu
from jax.experimental.pallas import tpu_sc as plsc

_NC = 2
_NS = 16
_NW = _NC * _NS
_CHUNK = 80
_NBUF = 4


def _sc_gather_pair(exe_idx, skill_idx, exe_table, skill_table):
    n = exe_idx.shape[0]
    d = exe_table.shape[1]
    per_w = n // _NW
    n_chunks = per_w // _CHUNK
    mesh = plsc.VectorSubcoreMesh(
        core_axis_name="c", subcore_axis_name="s",
        num_cores=_NC, num_subcores=_NS,
    )

    @functools.partial(
        pl.kernel,
        mesh=mesh,
        out_type=[
            jax.ShapeDtypeStruct((n, d), jnp.float32),
            jax.ShapeDtypeStruct((n, d), jnp.float32),
        ],
        scratch_types=[
            pltpu.VMEM((per_w,), jnp.int32),
            pltpu.VMEM((per_w,), jnp.int32),
            pltpu.VMEM((_NBUF, _CHUNK, d), jnp.float32),
            pltpu.VMEM((_NBUF, _CHUNK, d), jnp.float32),
            pltpu.SemaphoreType.DMA,
            pltpu.SemaphoreType.DMA,
        ],
    )
    def gather_kernel(exe_idx_hbm, skill_idx_hbm, exe_tab_hbm, skill_tab_hbm,
                      out_exe_hbm, out_skill_hbm,
                      idx_e, idx_s, rows_e, rows_s, sem_g, sem_w):
        wid = lax.axis_index("s") * _NC + lax.axis_index("c")
        base = wid * per_w
        pltpu.sync_copy(exe_idx_hbm.at[pl.ds(base, per_w)], idx_e)
        pltpu.sync_copy(skill_idx_hbm.at[pl.ds(base, per_w)], idx_s)
        n_groups = n_chunks // _NBUF

        def body(g, carry):
            gb = g * _NBUF * _CHUNK
            cps = []
            for k in range(_NBUF):
                io = gb + k * _CHUNK
                cps.append(pltpu.async_copy(
                    exe_tab_hbm.at[idx_e.at[pl.ds(io, _CHUNK)]],
                    rows_e.at[k], sem_g))
                cps.append(pltpu.async_copy(
                    skill_tab_hbm.at[idx_s.at[pl.ds(io, _CHUNK)]],
                    rows_s.at[k], sem_g))
            wcs = []
            for k in range(_NBUF):
                cps[2 * k].wait()
                cps[2 * k + 1].wait()
                off = base + gb + k * _CHUNK
                wcs.append(pltpu.async_copy(
                    rows_e.at[k], out_exe_hbm.at[pl.ds(off, _CHUNK)], sem_w))
                wcs.append(pltpu.async_copy(
                    rows_s.at[k], out_skill_hbm.at[pl.ds(off, _CHUNK)], sem_w))
            for w in wcs:
                w.wait()
            return carry

        lax.fori_loop(0, n_groups, body, 0)

    return gather_kernel(exe_idx, skill_idx, exe_table, skill_table)


def _tc_ed_body(seqs, bsz, comb, x, pos, W, b, etW, etb, resp, enc_o, dec_o):
    Wv = W[...]
    bv = b[...]
    etWv = etW[...]
    etbv = etb[...]
    respv = resp[...]
    d = bv.shape[1]
    dn = (((0,), (0,)), ((), ()))
    ones_row = jnp.ones((1, bsz), jnp.float32)
    ones128 = jnp.ones((1, d), jnp.float32)
    xw = jnp.dot(x[...], Wv, preferred_element_type=jnp.float32)
    for j in range(seqs):
        cj = comb[j]
        posr = pos[j]
        ids_row = (cj[0:1, :] + cj[1:2, :]).astype(jnp.float32)
        r_row = cj[2:3, :]
        et_row = lax.bitcast_convert_type(cj[3:4, :], jnp.float32)
        sl = pl.ds(j * bsz, bsz)
        a_enc = jnp.concatenate([ids_row, ones_row], axis=0)
        b_enc = jnp.concatenate([ones128, bv + posr], axis=0)
        enc_o[sl, :] = xw[j * bsz:(j + 1) * bsz, :] + lax.dot_general(
            a_enc, b_enc, dn, preferred_element_type=jnp.float32)
        oh = jnp.concatenate(
            [(r_row == t).astype(jnp.float32) for t in range(respv.shape[0])],
            axis=0,
        )
        a_dec = jnp.concatenate([oh, et_row, ones_row], axis=0)
        b_dec = jnp.concatenate([respv, etWv, etbv + posr], axis=0)
        dec_o[sl, :] = lax.dot_general(
            a_dec, b_dec, dn, preferred_element_type=jnp.float32)


def _tc_out_body(y, gexe, gskill, W, b, out_o):
    out_o[...] = (
        jnp.dot(y[...], W[...], preferred_element_type=jnp.float32)
        + b[...] + gexe[...] + gskill[...]
    )


def _tc_enc_dec(comb, x, pos, W, b, etW, etb, resp, seqs, interpret=False):
    n, nlp = x.shape
    d = W.shape[1]
    s_dim = pos.shape[0]
    bsz = n // s_dim
    rows = seqs * bsz
    grid = (n // rows,)
    row_spec = lambda w: pl.BlockSpec((rows, w), lambda i: (i, 0))
    full = lambda h, w: pl.BlockSpec((h, w), lambda i: (0, 0))
    return pl.pallas_call(
        functools.partial(_tc_ed_body, seqs, bsz),
        grid=grid,
        in_specs=[
            pl.BlockSpec((seqs, 4, bsz), lambda i: (i, 0, 0)),
            row_spec(nlp),
            pl.BlockSpec((seqs, 1, d), lambda i: (i, 0, 0)),
            full(nlp, d), full(1, d), full(1, d), full(1, d),
            full(resp.shape[0], d),
        ],
        out_specs=[row_spec(d), row_spec(d)],
        out_shape=[jax.ShapeDtypeStruct((n, d), jnp.float32)] * 2,
        compiler_params=pltpu.CompilerParams(
            dimension_semantics=("parallel",),
        ),
        interpret=interpret,
    )(comb, x, pos, W, b, etW, etb, resp)


def _tc_out(y, g_exe, g_skill, W, b, rows, interpret=False):
    n, nlp = y.shape
    d = W.shape[1]
    grid = (n // rows,)
    row_spec = lambda w: pl.BlockSpec((rows, w), lambda i: (i, 0))
    return pl.pallas_call(
        _tc_out_body,
        grid=grid,
        in_specs=[
            row_spec(nlp), row_spec(d), row_spec(d),
            pl.BlockSpec((nlp, d), lambda i: (0, 0)),
            pl.BlockSpec((1, d), lambda i: (0, 0)),
        ],
        out_specs=[row_spec(d)],
        out_shape=[jax.ShapeDtypeStruct((n, d), jnp.float32)],
        compiler_params=pltpu.CompilerParams(
            dimension_semantics=("parallel",),
        ),
        interpret=interpret,
    )(y, g_exe, g_skill, W, b)[0]


def kernel(input_nlp_embedding, input_exercise, input_skill, input_r,
           in_elapsed_time, output_nlp_embedding, out_exercise, out_skill,
           exercise_table, skill_table, response_table, pos_table,
           nlp_W, nlp_b, et_W, et_b):
    b_dim, s_dim, nlp = input_nlp_embedding.shape
    d = nlp_W.shape[1]
    n = b_dim * s_dim

    x_in = jnp.transpose(input_nlp_embedding, (1, 0, 2)).reshape(n, nlp)
    x_out = jnp.transpose(output_nlp_embedding, (1, 0, 2)).reshape(n, nlp)
    comb = jnp.concatenate([
        jnp.transpose(input_exercise).reshape(s_dim, 1, b_dim),
        jnp.transpose(input_skill).reshape(s_dim, 1, b_dim),
        jnp.transpose(input_r).reshape(s_dim, 1, b_dim),
        lax.bitcast_convert_type(
            jnp.transpose(in_elapsed_time, (1, 2, 0)), jnp.int32),
    ], axis=1)
    oexe_t = jnp.transpose(out_exercise).reshape(n)
    oskill_t = jnp.transpose(out_skill).reshape(n)

    g_exe, g_skill = _sc_gather_pair(
        oexe_t, oskill_t, exercise_table, skill_table,
    )

    enc2, dec2 = _tc_enc_dec(
        comb, x_in, pos_table.reshape(s_dim, 1, d),
        nlp_W, nlp_b.reshape(1, d), et_W, et_b.reshape(1, d),
        response_table, seqs=2,
    )
    out2 = _tc_out(x_out, g_exe, g_skill, nlp_W, nlp_b.reshape(1, d),
                   rows=2 * b_dim)

    def back(a2):
        return jnp.transpose(a2.reshape(s_dim, b_dim, d), (1, 0, 2))

    return (back(enc2), back(dec2), back(out2))
